# baseline (speedup 1.0000x reference)
import functools

import jax
import jax.numpy as jnp
from jax import lax
from jax.experimental import pallas as pl
from jax.experimental.pallas import tpu as pltpu
from jax.experimental.pallas import tpu_sc as plsc

BATCH = 4
SEQ = 8192
EMB = 1024
NUM_CORES = 2
NUM_SUBCORES = 16
NUM_WORKERS = NUM_CORES * NUM_SUBCORES
ROWS_PER_WORKER = SEQ // NUM_WORKERS
CHUNK_ROWS = 64
NUM_CHUNKS = ROWS_PER_WORKER // CHUNK_ROWS

SC_BATCH = 1
TC_BATCH = BATCH - SC_BATCH
BS = 512

_mesh = plsc.VectorSubcoreMesh(core_axis_name="c", subcore_axis_name="s")


@functools.partial(
    pl.kernel,
    mesh=_mesh,
    out_type=jax.ShapeDtypeStruct((SC_BATCH, SEQ, EMB), jnp.float32),
    scratch_types=[pltpu.VMEM((CHUNK_ROWS, EMB), jnp.float32)],
)
def _sc_broadcast(table_hbm, out_hbm, buf):
    wid = lax.axis_index("s") * NUM_CORES + lax.axis_index("c")
    base = wid * ROWS_PER_WORKER
    for i in range(NUM_CHUNKS):
        row = base + i * CHUNK_ROWS
        pltpu.sync_copy(table_hbm.at[pl.ds(row, CHUNK_ROWS)], buf)
        for b in range(SC_BATCH):
            pltpu.sync_copy(buf, out_hbm.at[b, pl.ds(row, CHUNK_ROWS)])


def _tc_body(tab_ref, out_ref):
    t = tab_ref[...]
    for b in range(TC_BATCH):
        out_ref[b] = t


_tc_call = pl.pallas_call(
    _tc_body,
    grid=(SEQ // BS,),
    in_specs=[pl.BlockSpec((BS, EMB), lambda i: (i, 0))],
    out_specs=pl.BlockSpec((TC_BATCH, BS, EMB), lambda i: (0, i, 0)),
    out_shape=jax.ShapeDtypeStruct((TC_BATCH, SEQ, EMB), jnp.float32),
)


def kernel(src, seg, table):
    del src, seg
    return (_sc_broadcast(table), _tc_call(table))


# speedup vs baseline: 1.1098x; 1.1098x over previous
import functools

import jax
import jax.numpy as jnp
from jax import lax
from jax.experimental import pallas as pl
from jax.experimental.pallas import tpu as pltpu
from jax.experimental.pallas import tpu_sc as plsc

BATCH = 4
SEQ = 8192
EMB = 1024
NUM_CORES = 2
NUM_SUBCORES = 16
NUM_WORKERS = NUM_CORES * NUM_SUBCORES
ROWS_PER_WORKER = SEQ // NUM_WORKERS
CHUNK_ROWS = 64
NUM_CHUNKS = ROWS_PER_WORKER // CHUNK_ROWS

_mesh = plsc.VectorSubcoreMesh(core_axis_name="c", subcore_axis_name="s")


@functools.partial(
    pl.kernel,
    mesh=_mesh,
    out_type=jax.ShapeDtypeStruct((BATCH, SEQ, EMB), jnp.float32),
    scratch_types=[pltpu.VMEM((CHUNK_ROWS, EMB), jnp.float32)],
)
def _broadcast_table(table_hbm, out_hbm, buf):
    wid = lax.axis_index("s") * NUM_CORES + lax.axis_index("c")
    base = wid * ROWS_PER_WORKER
    for i in range(NUM_CHUNKS):
        row = base + i * CHUNK_ROWS
        pltpu.sync_copy(table_hbm.at[pl.ds(row, CHUNK_ROWS)], buf)
        for b in range(BATCH):
            pltpu.sync_copy(buf, out_hbm.at[b, pl.ds(row, CHUNK_ROWS)])


def kernel(src, seg, table):
    del src, seg
    return _broadcast_table(table)
